# baseline (speedup 1.0000x reference)
import jax
import jax.numpy as jnp
from jax.experimental import pallas as pl
from jax.experimental.pallas import tpu as pltpu

_FIGSIZE = 416.0
_CONF_THRESH = 0.2
_B = 16
_N = 20000
_CHUNK = 2048
_GRID = (_N + _CHUNK - 1) // _CHUNK


def _body(thr_ref, bt_ref, gt_ref, loss_ref, probs_ref, sdet, scnt, sconf):
    i = pl.program_id(0)

    @pl.when(i == 0)
    def _init():
        sdet[...] = jnp.zeros_like(sdet)
        scnt[...] = jnp.zeros_like(scnt)
        sconf[...] = jnp.zeros_like(sconf)

    x = bt_ref[0]
    y = bt_ref[1]
    w = bt_ref[2]
    h = bt_ref[3]
    conf = bt_ref[4]
    cls_id = bt_ref[5]

    wh = w * 0.5
    hh = h * 0.5
    bx1 = (x - wh) * _FIGSIZE
    by1 = (y - hh) * _FIGSIZE
    bx2 = (x + wh) * _FIGSIZE
    by2 = (y + hh) * _FIGSIZE

    gx1 = gt_ref[:, 0:1]
    gy1 = gt_ref[:, 1:2]
    gx2 = gt_ref[:, 2:3]
    gy2 = gt_ref[:, 3:4]

    ix1 = jnp.maximum(bx1, gx1)
    iy1 = jnp.maximum(by1, gy1)
    ix2 = jnp.minimum(bx2, gx2)
    iy2 = jnp.minimum(by2, gy2)
    inter = jnp.clip(ix2 - ix1, 0.0) * jnp.clip(iy2 - iy1, 0.0)
    area_b = jnp.clip(bx2 - bx1, 0.0) * jnp.clip(by2 - by1, 0.0)
    area_g = (gx2 - gx1) * (gy2 - gy1)
    ious = inter / (area_b + area_g - inter + 1e-9)

    thr = thr_ref[0]
    lane = jax.lax.broadcasted_iota(jnp.int32, (_B, _CHUNK), 1)
    in_bounds = (i * _CHUNK + lane) < _N
    valid = (ious >= thr) & (cls_id == 0.0) & (conf > _CONF_THRESH) & in_bounds

    s = jnp.clip(conf, 1e-6, 1.0 - 1e-6)
    term = -jnp.log1p(-s) * ious

    zero = jnp.zeros_like(term)
    sdet[...] += jnp.sum(jnp.where(valid, term, zero), axis=1, keepdims=True)
    scnt[...] += jnp.sum(jnp.where(valid, 1.0, 0.0), axis=1, keepdims=True)
    sconf[...] += jnp.sum(jnp.where(valid, conf, zero), axis=1, keepdims=True)

    @pl.when(i == _GRID - 1)
    def _fin():
        det = sdet[...]
        cnt = scnt[...]
        sc = sconf[...]
        any_v = cnt > 0.0
        det_i = jnp.where(any_v, det, 0.0)
        probs_ref[...] = jnp.where(any_v, sc / jnp.maximum(cnt, 1.0), 0.0)
        loss_ref[...] = jnp.sum(det_i, keepdims=True) * (1.0 / _B)


def kernel(boxes, gt, iou_thresh):
    bt = jnp.transpose(boxes[..., jnp.array([0, 1, 2, 3, 4, 6])], (2, 0, 1))
    thr = jnp.reshape(jnp.asarray(iou_thresh, jnp.float32), (1,))

    loss, probs = pl.pallas_call(
        _body,
        grid=(_GRID,),
        in_specs=[
            pl.BlockSpec(memory_space=pltpu.SMEM),
            pl.BlockSpec((6, _B, _CHUNK), lambda i: (0, 0, i)),
            pl.BlockSpec((_B, 4), lambda i: (0, 0)),
        ],
        out_specs=[
            pl.BlockSpec((1, 1), lambda i: (0, 0)),
            pl.BlockSpec((_B, 1), lambda i: (0, 0)),
        ],
        out_shape=[
            jax.ShapeDtypeStruct((1, 1), jnp.float32),
            jax.ShapeDtypeStruct((_B, 1), jnp.float32),
        ],
        scratch_shapes=[
            pltpu.VMEM((_B, 1), jnp.float32),
            pltpu.VMEM((_B, 1), jnp.float32),
            pltpu.VMEM((_B, 1), jnp.float32),
        ],
    )(thr, bt, gt)
    return jnp.reshape(loss, ()), jnp.reshape(probs, (_B,))
